# baseline (speedup 1.0000x reference)
import functools

import jax
import jax.numpy as jnp
from jax import lax
from jax.experimental import pallas as pl
from jax.experimental.pallas import tpu as pltpu
from jax.experimental.pallas import tpu_sc as plsc

VOCAB = 1000000
D = 64
T = 200
B = 4096

NC = 2
NS = 16
NW = NC * NS

ROWS = B * T
RPW = ROWS // NW
CH = 128
NCH = RPW // CH

_mesh = plsc.VectorSubcoreMesh(core_axis_name="c", subcore_axis_name="s")


@functools.partial(
    pl.kernel,
    mesh=_mesh,
    out_type=jax.ShapeDtypeStruct((ROWS, D), jnp.float32),
    compiler_params=pltpu.CompilerParams(use_tc_tiling_on_sc=False),
    scratch_types=[
        pltpu.VMEM((CH,), jnp.int32),
        pltpu.VMEM((CH, D), jnp.float32),
        pltpu.VMEM((2 * T, D), jnp.float32),
        pltpu.SemaphoreType.DMA,
    ],
)
def _embed(x_hbm, tab_hbm, pos_hbm, out_hbm, idx_v, row_v, pos2_v, sem):
    wid = lax.axis_index("s") * NC + lax.axis_index("c")
    base = wid * RPW
    pltpu.sync_copy(pos_hbm, pos2_v.at[pl.ds(0, T)])
    pltpu.sync_copy(pos_hbm, pos2_v.at[pl.ds(T, T)])

    def chunk_body(c, carry):
        cb = base + c * CH
        t0 = lax.rem(cb, T)
        pltpu.sync_copy(x_hbm.at[pl.ds(cb, CH)], idx_v)
        pltpu.async_copy(tab_hbm.at[idx_v], row_v, sem).wait()

        def add_body(i, carry2):
            for j in range(D // 16):
                sl = pl.ds(j * 16, 16)
                row_v[i, sl] = row_v[i, sl] + pos2_v[t0 + i, sl]
            return carry2

        lax.fori_loop(0, CH, add_body, 0, unroll=4)
        pltpu.sync_copy(row_v, out_hbm.at[pl.ds(cb, CH)])
        return carry

    lax.fori_loop(0, NCH, chunk_body, 0)


def kernel(x, text_embedding, positional_embedding):
    xf = x.reshape(-1).astype(jnp.int32)
    out = _embed(xf, text_embedding, positional_embedding)
    return out.reshape(B, T, D)


# speedup vs baseline: 1.2054x; 1.2054x over previous
import functools

import jax
import jax.numpy as jnp
from jax import lax
from jax.experimental import pallas as pl
from jax.experimental.pallas import tpu as pltpu
from jax.experimental.pallas import tpu_sc as plsc

VOCAB = 1000000
D = 64
T = 200
B = 4096

NC = 2
NS = 16
NW = NC * NS

ROWS = B * T
RPW = ROWS // NW
CH = 128
NCH = RPW // CH
NBUF = 8

_mesh = plsc.VectorSubcoreMesh(core_axis_name="c", subcore_axis_name="s")


@functools.partial(
    pl.kernel,
    mesh=_mesh,
    out_type=jax.ShapeDtypeStruct((ROWS, D), jnp.float32),
    compiler_params=pltpu.CompilerParams(use_tc_tiling_on_sc=False),
    scratch_types=[
        pltpu.VMEM((NBUF, CH), jnp.int32),
        pltpu.VMEM((NBUF, CH, D), jnp.float32),
        pltpu.VMEM((2 * T, D), jnp.float32),
        pltpu.SemaphoreType.DMA((NBUF,)),
        pltpu.SemaphoreType.DMA((NBUF,)),
        pltpu.SemaphoreType.DMA((NBUF,)),
    ],
)
def _embed(x_hbm, tab_hbm, pos_hbm, out_hbm,
           idx_v, row_v, pos2_v, sem_i, sem_g, sem_o):
    wid = lax.axis_index("s") * NC + lax.axis_index("c")
    base = wid * RPW
    pltpu.sync_copy(pos_hbm, pos2_v.at[pl.ds(0, T)])
    pltpu.sync_copy(pos_hbm, pos2_v.at[pl.ds(T, T)])

    def group_body(g):
        for b in range(NBUF):
            cb = base + (g + b) * CH
            pltpu.async_copy(x_hbm.at[pl.ds(cb, CH)], idx_v.at[b],
                             sem_i.at[b])
        for b in range(NBUF):
            cb = base + (g + b) * CH
            pltpu.make_async_copy(x_hbm.at[pl.ds(cb, CH)], idx_v.at[b],
                                  sem_i.at[b]).wait()
            pltpu.async_copy(tab_hbm.at[idx_v.at[b]], row_v.at[b],
                             sem_g.at[b])
        for b in range(NBUF):
            cb = base + (g + b) * CH
            t0 = lax.rem(cb, T)
            pltpu.make_async_copy(tab_hbm.at[idx_v.at[b]], row_v.at[b],
                                  sem_g.at[b]).wait()

            def add_body(i, carry, b=b, t0=t0):
                for j in range(D // 16):
                    sl = pl.ds(j * 16, 16)
                    plsc.addupdate(row_v.at[b, i, sl], pos2_v[t0 + i, sl])
                return carry

            lax.fori_loop(0, CH, add_body, 0, unroll=4)
            pltpu.async_copy(row_v.at[b], out_hbm.at[pl.ds(cb, CH)],
                             sem_o.at[b])
        for b in range(NBUF):
            cb = base + (g + b) * CH
            pltpu.make_async_copy(row_v.at[b], out_hbm.at[pl.ds(cb, CH)],
                                  sem_o.at[b]).wait()

    pl.loop(0, NCH, step=NBUF)(group_body)


def kernel(x, text_embedding, positional_embedding):
    xf = x.reshape(-1).astype(jnp.int32)
    out = _embed(xf, text_embedding, positional_embedding)
    return out.reshape(B, T, D)
